# Initial kernel scaffold; baseline (speedup 1.0000x reference)
import jax
import jax.numpy as jnp
from jax.experimental import pallas as pl


def kernel(edge_index, node_emb, W1, b1, W2, b2, W3, b3):
    raise NotImplementedError("write your pallas kernel here")


# speedup vs baseline: 13.7098x; 13.7098x over previous
import functools

import jax
import jax.numpy as jnp
from jax import lax
from jax.experimental import pallas as pl
from jax.experimental.pallas import tpu as pltpu
from jax.experimental.pallas import tpu_sc as plsc

N = 10000
H = 128
E = 320000
NP = 10240
EP = 327680
K = 128
EPT = EP // 32
NCH = EPT // K
RPT = NP // 16
ZR = 64

_mesh = plsc.VectorSubcoreMesh(core_axis_name="c", subcore_axis_name="s")


@functools.partial(
    pl.kernel,
    out_type=jax.ShapeDtypeStruct((2, NP, 16), jnp.float32),
    mesh=_mesh,
    scratch_types=[
        pltpu.VMEM_SHARED((NP, 16), jnp.float32),
        pltpu.VMEM((ZR, 16), jnp.float32),
        pltpu.VMEM((K, 16), jnp.float32),
        pltpu.VMEM((1, K), jnp.int32),
    ],
)
def _deg_kernel(dst_hbm, out_hbm, acc, zbuf, obuf, didx):
    c = lax.axis_index("c")
    s = lax.axis_index("s")
    wid = c * 16 + s

    def zfill(i, _):
        zbuf[i, pl.ds(0, 16)] = jnp.zeros((16,), jnp.float32)
        return 0

    lax.fori_loop(0, ZR, zfill, 0)

    def ofill(i, _):
        obuf[i, pl.ds(0, 16)] = jnp.ones((16,), jnp.float32)
        return 0

    lax.fori_loop(0, K, ofill, 0)

    def zcopy(z, _):
        pltpu.sync_copy(zbuf, acc.at[pl.ds(s * RPT + z * ZR, ZR)])
        return 0

    lax.fori_loop(0, RPT // ZR, zcopy, 0)
    plsc.subcore_barrier()

    e0 = wid * EPT

    def chunk(g, _):
        pltpu.sync_copy(dst_hbm.at[pl.ds(e0 + g * K, K)], didx.at[0])
        pltpu.sync_copy(obuf, acc.at[didx.at[0]], add=True)
        return 0

    lax.fori_loop(0, NCH, chunk, 0)
    plsc.subcore_barrier()
    pltpu.sync_copy(acc.at[pl.ds(s * RPT, RPT)], out_hbm.at[c, pl.ds(s * RPT, RPT)])


@functools.partial(
    pl.kernel,
    out_type=jax.ShapeDtypeStruct((2, NP, H), jnp.float32),
    mesh=_mesh,
    scratch_types=[
        pltpu.VMEM_SHARED((NP, H), jnp.float32),
        pltpu.VMEM((ZR, H), jnp.float32),
        pltpu.VMEM((1, K), jnp.int32),
        pltpu.VMEM((1, K), jnp.int32),
        pltpu.VMEM((K, H), jnp.float32),
        pltpu.SemaphoreType.DMA,
    ],
)
def _prop_kernel(src_hbm, dst_hbm, y_hbm, out_hbm, acc, zbuf, sidx, didx, rows, sem):
    c = lax.axis_index("c")
    s = lax.axis_index("s")
    wid = c * 16 + s

    def zfill(i, _):
        zbuf[i // 8, pl.ds((i % 8) * 16, 16)] = jnp.zeros((16,), jnp.float32)
        return 0

    lax.fori_loop(0, ZR * (H // 16), zfill, 0)

    def zcopy(z, _):
        pltpu.sync_copy(zbuf, acc.at[pl.ds(s * RPT + z * ZR, ZR)])
        return 0

    lax.fori_loop(0, RPT // ZR, zcopy, 0)
    plsc.subcore_barrier()

    e0 = wid * EPT

    def chunk(g, _):
        base = e0 + g * K
        pltpu.sync_copy(src_hbm.at[pl.ds(base, K)], sidx.at[0])
        pltpu.sync_copy(dst_hbm.at[pl.ds(base, K)], didx.at[0])
        pltpu.async_copy(y_hbm.at[sidx.at[0]], rows, sem).wait()
        pltpu.sync_copy(rows, acc.at[didx.at[0]], add=True)
        return 0

    lax.fori_loop(0, NCH, chunk, 0)
    plsc.subcore_barrier()
    pltpu.sync_copy(acc.at[pl.ds(s * RPT, RPT)], out_hbm.at[c, pl.ds(s * RPT, RPT)])


_B = 1024


def _t1_body(deg_ref, x_ref, w_ref, y_ref, dinv_ref):
    i = pl.program_id(0)
    deg = deg_ref[0, :, 0:1] + deg_ref[1, :, 0:1] + 1.0
    row = i * _B + lax.broadcasted_iota(jnp.int32, (_B, 1), 0)
    dinv = jnp.where(row < N, 1.0 / jnp.sqrt(deg), 0.0)
    dinv_ref[...] = dinv
    y_ref[...] = jnp.dot(x_ref[...], w_ref[...],
                         preferred_element_type=jnp.float32) * dinv


def _tmid_body(agg_ref, y_ref, dinv_ref, b_ref, w_ref, out_ref):
    dinv = dinv_ref[...]
    h = (agg_ref[0] + agg_ref[1] + y_ref[...]) * dinv + b_ref[...]
    x = jnp.maximum(h, 0.0)
    out_ref[...] = jnp.dot(x, w_ref[...],
                           preferred_element_type=jnp.float32) * dinv


def _t4_body(agg_ref, y_ref, dinv_ref, b_ref, out_ref):
    out_ref[...] = ((agg_ref[0] + agg_ref[1] + y_ref[...]) * dinv_ref[...]
                    + b_ref[...])


_t1 = pl.pallas_call(
    _t1_body,
    grid=(NP // _B,),
    in_specs=[
        pl.BlockSpec((2, _B, 16), lambda i: (0, i, 0)),
        pl.BlockSpec((_B, H), lambda i: (i, 0)),
        pl.BlockSpec((H, H), lambda i: (0, 0)),
    ],
    out_specs=[
        pl.BlockSpec((_B, H), lambda i: (i, 0)),
        pl.BlockSpec((_B, 1), lambda i: (i, 0)),
    ],
    out_shape=[
        jax.ShapeDtypeStruct((NP, H), jnp.float32),
        jax.ShapeDtypeStruct((NP, 1), jnp.float32),
    ],
)

_tmid = pl.pallas_call(
    _tmid_body,
    grid=(NP // _B,),
    in_specs=[
        pl.BlockSpec((2, _B, H), lambda i: (0, i, 0)),
        pl.BlockSpec((_B, H), lambda i: (i, 0)),
        pl.BlockSpec((_B, 1), lambda i: (i, 0)),
        pl.BlockSpec((1, H), lambda i: (0, 0)),
        pl.BlockSpec((H, H), lambda i: (0, 0)),
    ],
    out_specs=pl.BlockSpec((_B, H), lambda i: (i, 0)),
    out_shape=jax.ShapeDtypeStruct((NP, H), jnp.float32),
)

_t4 = pl.pallas_call(
    _t4_body,
    grid=(NP // _B,),
    in_specs=[
        pl.BlockSpec((2, _B, H), lambda i: (0, i, 0)),
        pl.BlockSpec((_B, H), lambda i: (i, 0)),
        pl.BlockSpec((_B, 1), lambda i: (i, 0)),
        pl.BlockSpec((1, H), lambda i: (0, 0)),
    ],
    out_specs=pl.BlockSpec((_B, H), lambda i: (i, 0)),
    out_shape=jax.ShapeDtypeStruct((NP, H), jnp.float32),
)


def kernel(edge_index, node_emb, W1, b1, W2, b2, W3, b3):
    src = edge_index[0].astype(jnp.int32)
    dst = edge_index[1].astype(jnp.int32)
    pad = N + (jnp.arange(EP - E, dtype=jnp.int32) % (NP - N))
    srcp = jnp.concatenate([src, pad])
    dstp = jnp.concatenate([dst, pad])
    x0 = jnp.pad(node_emb, ((0, NP - N), (0, 0)))

    degs = _deg_kernel(dstp)
    y1, dinv = _t1(degs, x0, W1)
    agg1 = _prop_kernel(srcp, dstp, y1)
    y2 = _tmid(agg1, y1, dinv, b1.reshape(1, H), W2)
    agg2 = _prop_kernel(srcp, dstp, y2)
    y3 = _tmid(agg2, y2, dinv, b2.reshape(1, H), W3)
    agg3 = _prop_kernel(srcp, dstp, y3)
    out = _t4(agg3, y3, dinv, b3.reshape(1, H))
    return out[:N]


# speedup vs baseline: 24.8535x; 1.8128x over previous
import functools

import jax
import jax.numpy as jnp
from jax import lax
from jax.experimental import pallas as pl
from jax.experimental.pallas import tpu as pltpu
from jax.experimental.pallas import tpu_sc as plsc

N = 10000
H = 128
E = 320000
NP = 10240
EP = 327680
K = 128
EPT = EP // 32
NCH = EPT // K
RPT = NP // 16
ZR = 64

_mesh = plsc.VectorSubcoreMesh(core_axis_name="c", subcore_axis_name="s")


@functools.partial(
    pl.kernel,
    out_type=jax.ShapeDtypeStruct((2, NP, 16), jnp.float32),
    mesh=_mesh,
    scratch_types=[
        pltpu.VMEM_SHARED((NP, 16), jnp.float32),
        pltpu.VMEM((ZR, 16), jnp.float32),
        pltpu.VMEM((K, 16), jnp.float32),
        pltpu.VMEM((2, K), jnp.int32),
        pltpu.SemaphoreType.DMA,
        pltpu.SemaphoreType.DMA,
    ],
)
def _deg_kernel(dst_hbm, out_hbm, acc, zbuf, obuf, didx, isem0, isem1):
    isems = [isem0, isem1]
    c = lax.axis_index("c")
    s = lax.axis_index("s")
    wid = c * 16 + s

    def zfill(i, _):
        zbuf[i, pl.ds(0, 16)] = jnp.zeros((16,), jnp.float32)
        return 0

    lax.fori_loop(0, ZR, zfill, 0)

    def ofill(i, _):
        obuf[i, pl.ds(0, 16)] = jnp.ones((16,), jnp.float32)
        return 0

    lax.fori_loop(0, K, ofill, 0)

    def zcopy(z, _):
        pltpu.sync_copy(zbuf, acc.at[pl.ds(s * RPT + z * ZR, ZR)])
        return 0

    lax.fori_loop(0, RPT // ZR, zcopy, 0)
    plsc.subcore_barrier()

    for b in range(2):
        pltpu.async_copy(dst_hbm.at[wid, b], didx.at[b], isems[b])

    def chunk(t, _):
        for b in range(2):
            g = t * 2 + b
            pltpu.make_async_copy(dst_hbm.at[wid, g], didx.at[b], isems[b]).wait()
            pltpu.sync_copy(obuf, acc.at[didx.at[b]], add=True)
            pltpu.async_copy(dst_hbm.at[wid, g + 2], didx.at[b], isems[b])
        return 0

    lax.fori_loop(0, NCH // 2 - 1, chunk, 0)
    for b in range(2):
        g = NCH - 2 + b
        pltpu.make_async_copy(dst_hbm.at[wid, g], didx.at[b], isems[b]).wait()
        pltpu.sync_copy(obuf, acc.at[didx.at[b]], add=True)
    plsc.subcore_barrier()
    pltpu.sync_copy(acc.at[pl.ds(s * RPT, RPT)], out_hbm.at[c, pl.ds(s * RPT, RPT)])


_R = 2
_ZB = 16


@functools.partial(
    pl.kernel,
    out_type=jax.ShapeDtypeStruct((2, NP, H), jnp.float32),
    mesh=_mesh,
    scratch_types=[
        pltpu.VMEM_SHARED((NP, H), jnp.float32),
        pltpu.VMEM((_ZB, H), jnp.float32),
        pltpu.VMEM((_R, 2, K), jnp.int32),
        pltpu.VMEM((_R, K, H), jnp.float32),
        pltpu.SemaphoreType.DMA,
        pltpu.SemaphoreType.DMA,
    ],
)
def _prop_kernel(ed_hbm, y_hbm, out_hbm, acc, zbuf, eidx, rows, sem0, sem1):
    sems = [sem0, sem1]
    c = lax.axis_index("c")
    s = lax.axis_index("s")
    wid = c * 16 + s

    def zfill(i, _):
        zbuf[i // 8, pl.ds((i % 8) * 16, 16)] = jnp.zeros((16,), jnp.float32)
        return 0

    lax.fori_loop(0, _ZB * (H // 16), zfill, 0)

    def zcopy(z, _):
        pltpu.sync_copy(zbuf, acc.at[pl.ds(s * RPT + z * _ZB, _ZB)])
        return 0

    lax.fori_loop(0, RPT // _ZB, zcopy, 0)
    plsc.subcore_barrier()

    for b in range(_R):
        pltpu.sync_copy(ed_hbm.at[wid, b], eidx.at[b])
        pltpu.async_copy(y_hbm.at[eidx.at[b, 0]], rows.at[b], sems[b])

    def outer(t, _):
        for b in range(_R):
            g = t * _R + b
            pltpu.make_async_copy(y_hbm.at[eidx.at[b, 0]], rows.at[b], sems[b]).wait()
            pltpu.sync_copy(rows.at[b], acc.at[eidx.at[b, 1]], add=True)
            pltpu.sync_copy(ed_hbm.at[wid, g + _R], eidx.at[b])
            pltpu.async_copy(y_hbm.at[eidx.at[b, 0]], rows.at[b], sems[b])
        return 0

    lax.fori_loop(0, NCH // _R - 1, outer, 0)
    for b in range(_R):
        pltpu.make_async_copy(y_hbm.at[eidx.at[b, 0]], rows.at[b], sems[b]).wait()
        pltpu.sync_copy(rows.at[b], acc.at[eidx.at[b, 1]], add=True)
    plsc.subcore_barrier()
    pltpu.sync_copy(acc.at[pl.ds(s * RPT, RPT)], out_hbm.at[c, pl.ds(s * RPT, RPT)])


_B = 1024


def _t1_body(deg_ref, x_ref, w_ref, y_ref, dinv_ref):
    i = pl.program_id(0)
    deg = deg_ref[0, :, 0:1] + deg_ref[1, :, 0:1] + 1.0
    row = i * _B + lax.broadcasted_iota(jnp.int32, (_B, 1), 0)
    dinv = jnp.where(row < N, 1.0 / jnp.sqrt(deg), 0.0)
    dinv_ref[...] = dinv
    y_ref[...] = jnp.dot(x_ref[...], w_ref[...],
                         preferred_element_type=jnp.float32) * dinv


def _tmid_body(agg_ref, y_ref, dinv_ref, b_ref, w_ref, out_ref):
    dinv = dinv_ref[...]
    h = (agg_ref[0] + agg_ref[1] + y_ref[...]) * dinv + b_ref[...]
    x = jnp.maximum(h, 0.0)
    out_ref[...] = jnp.dot(x, w_ref[...],
                           preferred_element_type=jnp.float32) * dinv


def _t4_body(agg_ref, y_ref, dinv_ref, b_ref, out_ref):
    out_ref[...] = ((agg_ref[0] + agg_ref[1] + y_ref[...]) * dinv_ref[...]
                    + b_ref[...])


_t1 = pl.pallas_call(
    _t1_body,
    grid=(NP // _B,),
    in_specs=[
        pl.BlockSpec((2, _B, 16), lambda i: (0, i, 0)),
        pl.BlockSpec((_B, H), lambda i: (i, 0)),
        pl.BlockSpec((H, H), lambda i: (0, 0)),
    ],
    out_specs=[
        pl.BlockSpec((_B, H), lambda i: (i, 0)),
        pl.BlockSpec((_B, 1), lambda i: (i, 0)),
    ],
    out_shape=[
        jax.ShapeDtypeStruct((NP, H), jnp.float32),
        jax.ShapeDtypeStruct((NP, 1), jnp.float32),
    ],
)

_tmid = pl.pallas_call(
    _tmid_body,
    grid=(NP // _B,),
    in_specs=[
        pl.BlockSpec((2, _B, H), lambda i: (0, i, 0)),
        pl.BlockSpec((_B, H), lambda i: (i, 0)),
        pl.BlockSpec((_B, 1), lambda i: (i, 0)),
        pl.BlockSpec((1, H), lambda i: (0, 0)),
        pl.BlockSpec((H, H), lambda i: (0, 0)),
    ],
    out_specs=pl.BlockSpec((_B, H), lambda i: (i, 0)),
    out_shape=jax.ShapeDtypeStruct((NP, H), jnp.float32),
)

_t4 = pl.pallas_call(
    _t4_body,
    grid=(NP // _B,),
    in_specs=[
        pl.BlockSpec((2, _B, H), lambda i: (0, i, 0)),
        pl.BlockSpec((_B, H), lambda i: (i, 0)),
        pl.BlockSpec((_B, 1), lambda i: (i, 0)),
        pl.BlockSpec((1, H), lambda i: (0, 0)),
    ],
    out_specs=pl.BlockSpec((_B, H), lambda i: (i, 0)),
    out_shape=jax.ShapeDtypeStruct((NP, H), jnp.float32),
)


def kernel(edge_index, node_emb, W1, b1, W2, b2, W3, b3):
    src = edge_index[0].astype(jnp.int32)
    dst = edge_index[1].astype(jnp.int32)
    pad = N + (jnp.arange(EP - E, dtype=jnp.int32) % (NP - N))
    srcp = jnp.concatenate([src, pad]).reshape(32, NCH, 1, K)
    dstp = jnp.concatenate([dst, pad]).reshape(32, NCH, 1, K)
    ed = jnp.concatenate([srcp, dstp], axis=2)
    x0 = jnp.pad(node_emb, ((0, NP - N), (0, 0)))

    degs = _deg_kernel(dstp.reshape(32, NCH, K))
    y1, dinv = _t1(degs, x0, W1)
    agg1 = _prop_kernel(ed, y1)
    y2 = _tmid(agg1, y1, dinv, b1.reshape(1, H), W2)
    agg2 = _prop_kernel(ed, y2)
    y3 = _tmid(agg2, y2, dinv, b2.reshape(1, H), W3)
    agg3 = _prop_kernel(ed, y3)
    out = _t4(agg3, y3, dinv, b3.reshape(1, H))
    return out[:N]


# speedup vs baseline: 27.1676x; 1.0931x over previous
import functools

import jax
import jax.numpy as jnp
from jax import lax
from jax.experimental import pallas as pl
from jax.experimental.pallas import tpu as pltpu
from jax.experimental.pallas import tpu_sc as plsc

N = 10000
H = 128
E = 320000
NP = 10240
EP = 327680
K = 128
EPT = EP // 32
NCH = EPT // K
RPT = NP // 16
ZR = 64

_mesh = plsc.VectorSubcoreMesh(core_axis_name="c", subcore_axis_name="s")


@functools.partial(
    pl.kernel,
    out_type=jax.ShapeDtypeStruct((2, NP, 16), jnp.float32),
    mesh=_mesh,
    scratch_types=[
        pltpu.VMEM_SHARED((NP, 16), jnp.float32),
        pltpu.VMEM((ZR, 16), jnp.float32),
        pltpu.VMEM((K, 16), jnp.float32),
        pltpu.VMEM((2, K), jnp.int32),
        pltpu.SemaphoreType.DMA,
        pltpu.SemaphoreType.DMA,
    ],
)
def _deg_kernel(dst_hbm, out_hbm, acc, zbuf, obuf, didx, isem0, isem1):
    isems = [isem0, isem1]
    c = lax.axis_index("c")
    s = lax.axis_index("s")
    wid = c * 16 + s

    def zfill(i, _):
        zbuf[i, pl.ds(0, 16)] = jnp.zeros((16,), jnp.float32)
        return 0

    lax.fori_loop(0, ZR, zfill, 0)

    def ofill(i, _):
        obuf[i, pl.ds(0, 16)] = jnp.ones((16,), jnp.float32)
        return 0

    lax.fori_loop(0, K, ofill, 0)

    def zcopy(z, _):
        pltpu.sync_copy(zbuf, acc.at[pl.ds(s * RPT + z * ZR, ZR)])
        return 0

    lax.fori_loop(0, RPT // ZR, zcopy, 0)
    plsc.subcore_barrier()

    for b in range(2):
        pltpu.async_copy(dst_hbm.at[wid, b], didx.at[b], isems[b])

    def chunk(t, _):
        for b in range(2):
            g = t * 2 + b
            pltpu.make_async_copy(dst_hbm.at[wid, g], didx.at[b], isems[b]).wait()
            pltpu.sync_copy(obuf, acc.at[didx.at[b]], add=True)
            pltpu.async_copy(dst_hbm.at[wid, g + 2], didx.at[b], isems[b])
        return 0

    lax.fori_loop(0, NCH // 2 - 1, chunk, 0)
    for b in range(2):
        g = NCH - 2 + b
        pltpu.make_async_copy(dst_hbm.at[wid, g], didx.at[b], isems[b]).wait()
        pltpu.sync_copy(obuf, acc.at[didx.at[b]], add=True)
    plsc.subcore_barrier()
    pltpu.sync_copy(acc.at[pl.ds(s * RPT, RPT)], out_hbm.at[c, pl.ds(s * RPT, RPT)])


_R = 2
_RI = 5


@functools.partial(
    pl.kernel,
    out_type=jax.ShapeDtypeStruct((2, NP, H), jnp.float32),
    mesh=_mesh,
    scratch_types=[
        pltpu.VMEM_SHARED((NP, H), jnp.float32),
        pltpu.VMEM((_RI, 2, K), jnp.int32),
        pltpu.VMEM((_R, K, H), jnp.float32),
        [pltpu.SemaphoreType.DMA] * _R,
        [pltpu.SemaphoreType.DMA] * _R,
        [pltpu.SemaphoreType.DMA] * _RI,
    ],
)
def _prop_kernel(ed_hbm, y_hbm, z_hbm, out_hbm, acc, eidx, rows, gsems, ssems, isems):
    c = lax.axis_index("c")
    s = lax.axis_index("s")
    wid = c * 16 + s

    pltpu.sync_copy(z_hbm, acc.at[pl.ds(s * RPT, RPT)])
    plsc.subcore_barrier()

    for j in range(4):
        pltpu.async_copy(ed_hbm.at[wid, j], eidx.at[j], isems[j])
    pltpu.make_async_copy(ed_hbm.at[wid, 0], eidx.at[0], isems[0]).wait()
    pltpu.async_copy(y_hbm.at[eidx.at[0, 0]], rows.at[0], gsems[0])

    def outer(t, _):
        for k in range(10):
            g = t * 10 + k
            r = k % _R
            j = k % _RI
            rp = (k + 1) % _R
            jn = (k + 1) % _RI
            jm = (k + 4) % _RI

            @pl.when(jnp.logical_and(g >= 1, g <= NCH))
            def _():
                pltpu.make_async_copy(rows.at[rp], acc.at[eidx.at[jm, 1]],
                                      ssems[rp]).wait()

            @pl.when(g < NCH - 1)
            def _():
                pltpu.make_async_copy(ed_hbm.at[wid, g + 1], eidx.at[jn],
                                      isems[jn]).wait()
                pltpu.async_copy(y_hbm.at[eidx.at[jn, 0]], rows.at[rp],
                                 gsems[rp])

            @pl.when(g < NCH)
            def _():
                pltpu.make_async_copy(y_hbm.at[eidx.at[j, 0]], rows.at[r],
                                      gsems[r]).wait()
                pltpu.async_copy(rows.at[r], acc.at[eidx.at[j, 1]], ssems[r],
                                 add=True)

            @pl.when(g < NCH - 4)
            def _():
                pltpu.async_copy(ed_hbm.at[wid, g + 4], eidx.at[jm], isems[jm])

        return 0

    lax.fori_loop(0, (NCH + 1 + 9) // 10, outer, 0)
    plsc.subcore_barrier()
    pltpu.sync_copy(acc.at[pl.ds(s * RPT, RPT)],
                    out_hbm.at[c, pl.ds(s * RPT, RPT)])


_B = 1024


def _t1_body(deg_ref, x_ref, w_ref, y_ref, dinv_ref):
    i = pl.program_id(0)
    deg = deg_ref[0, :, 0:1] + deg_ref[1, :, 0:1] + 1.0
    row = i * _B + lax.broadcasted_iota(jnp.int32, (_B, 1), 0)
    dinv = jnp.where(row < N, 1.0 / jnp.sqrt(deg), 0.0)
    dinv_ref[...] = dinv
    y_ref[...] = jnp.dot(x_ref[...], w_ref[...],
                         preferred_element_type=jnp.float32) * dinv


def _tmid_body(agg_ref, y_ref, dinv_ref, b_ref, w_ref, out_ref):
    dinv = dinv_ref[...]
    h = (agg_ref[0] + agg_ref[1] + y_ref[...]) * dinv + b_ref[...]
    x = jnp.maximum(h, 0.0)
    out_ref[...] = jnp.dot(x, w_ref[...],
                           preferred_element_type=jnp.float32) * dinv


def _t4_body(agg_ref, y_ref, dinv_ref, b_ref, out_ref):
    out_ref[...] = ((agg_ref[0] + agg_ref[1] + y_ref[...]) * dinv_ref[...]
                    + b_ref[...])


_t1 = pl.pallas_call(
    _t1_body,
    grid=(NP // _B,),
    in_specs=[
        pl.BlockSpec((2, _B, 16), lambda i: (0, i, 0)),
        pl.BlockSpec((_B, H), lambda i: (i, 0)),
        pl.BlockSpec((H, H), lambda i: (0, 0)),
    ],
    out_specs=[
        pl.BlockSpec((_B, H), lambda i: (i, 0)),
        pl.BlockSpec((_B, 1), lambda i: (i, 0)),
    ],
    out_shape=[
        jax.ShapeDtypeStruct((NP, H), jnp.float32),
        jax.ShapeDtypeStruct((NP, 1), jnp.float32),
    ],
)

_tmid = pl.pallas_call(
    _tmid_body,
    grid=(NP // _B,),
    in_specs=[
        pl.BlockSpec((2, _B, H), lambda i: (0, i, 0)),
        pl.BlockSpec((_B, H), lambda i: (i, 0)),
        pl.BlockSpec((_B, 1), lambda i: (i, 0)),
        pl.BlockSpec((1, H), lambda i: (0, 0)),
        pl.BlockSpec((H, H), lambda i: (0, 0)),
    ],
    out_specs=pl.BlockSpec((_B, H), lambda i: (i, 0)),
    out_shape=jax.ShapeDtypeStruct((NP, H), jnp.float32),
)

_t4 = pl.pallas_call(
    _t4_body,
    grid=(NP // _B,),
    in_specs=[
        pl.BlockSpec((2, _B, H), lambda i: (0, i, 0)),
        pl.BlockSpec((_B, H), lambda i: (i, 0)),
        pl.BlockSpec((_B, 1), lambda i: (i, 0)),
        pl.BlockSpec((1, H), lambda i: (0, 0)),
    ],
    out_specs=pl.BlockSpec((_B, H), lambda i: (i, 0)),
    out_shape=jax.ShapeDtypeStruct((NP, H), jnp.float32),
)


def kernel(edge_index, node_emb, W1, b1, W2, b2, W3, b3):
    src = edge_index[0].astype(jnp.int32)
    dst = edge_index[1].astype(jnp.int32)
    pad = N + (jnp.arange(EP - E, dtype=jnp.int32) % (NP - N))
    srcp = jnp.concatenate([src, pad]).reshape(32, NCH, 1, K)
    dstp = jnp.concatenate([dst, pad]).reshape(32, NCH, 1, K)
    ed = jnp.concatenate([srcp, dstp], axis=2)
    zrows = jnp.zeros((640, H), jnp.float32)
    x0 = jnp.pad(node_emb, ((0, NP - N), (0, 0)))

    degs = _deg_kernel(dstp.reshape(32, NCH, K))
    y1, dinv = _t1(degs, x0, W1)
    agg1 = _prop_kernel(ed, y1, zrows)
    y2 = _tmid(agg1, y1, dinv, b1.reshape(1, H), W2)
    agg2 = _prop_kernel(ed, y2, zrows)
    y3 = _tmid(agg2, y2, dinv, b2.reshape(1, H), W3)
    agg3 = _prop_kernel(ed, y3, zrows)
    out = _t4(agg3, y3, dinv, b3.reshape(1, H))
    return out[:N]
